# baseline (speedup 1.0000x reference)
import functools

import jax
import jax.numpy as jnp
from jax import lax
from jax.experimental import pallas as pl
from jax.experimental.pallas import tpu as pltpu
from jax.experimental.pallas import tpu_sc as plsc

_NUM_WORKERS = 32
_LANES = 16


def _species_lookup_body(chunk, n, num_species, a_hbm, species_hbm,
                         out_hbm, a_v, out_v, spec_v, lut_v,
                         sem_s, sem_a0, sem_a1, sem_o0, sem_o1):
  wid = lax.axis_index("s") * 2 + lax.axis_index("c")
  is_last = wid == _NUM_WORKERS - 1
  base = jnp.minimum(wid * chunk, n - chunk)

  half = chunk // 2
  cp0 = pltpu.async_copy(a_hbm.at[pl.ds(base, half)],
                         a_v.at[pl.ds(0, half)], sem_a0)
  cp1 = pltpu.async_copy(a_hbm.at[pl.ds(base + half, half)],
                         a_v.at[pl.ds(half, half)], sem_a1)

  spec_v[...] = jnp.zeros((_LANES,), jnp.int32)
  cps = pltpu.async_copy(species_hbm, spec_v.at[pl.ds(0, num_species)],
                         sem_s)
  cps.wait()
  lane = lax.iota(jnp.int32, _LANES)
  plsc.store_scatter(lut_v, [spec_v[...]], lane, mask=lane < num_species)

  cp0.wait()

  @plsc.parallel_loop(0, half, _LANES, unroll=8)
  def _(off):
    out_v[pl.ds(off, _LANES)] = plsc.load_gather(lut_v,
                                                 [a_v[pl.ds(off, _LANES)]])

  skip = chunk - (n - (_NUM_WORKERS - 1) * chunk)

  @pl.when(jnp.logical_not(is_last))
  def _():
    pltpu.async_copy(out_v.at[pl.ds(0, half)],
                     out_hbm.at[pl.ds(base, half)], sem_o0)

  @pl.when(is_last)
  def _():
    pltpu.async_copy(out_v.at[pl.ds(skip, half - skip)],
                     out_hbm.at[pl.ds(base + skip, half - skip)], sem_o0)

  cp1.wait()

  @plsc.parallel_loop(half, chunk, _LANES, unroll=8)
  def _(off):
    out_v[pl.ds(off, _LANES)] = plsc.load_gather(lut_v,
                                                 [a_v[pl.ds(off, _LANES)]])

  co1 = pltpu.async_copy(out_v.at[pl.ds(half, half)],
                         out_hbm.at[pl.ds(base + half, half)], sem_o1)

  @pl.when(jnp.logical_not(is_last))
  def _():
    pltpu.make_async_copy(out_v.at[pl.ds(0, half)],
                          out_hbm.at[pl.ds(base, half)], sem_o0).wait()

  @pl.when(is_last)
  def _():
    pltpu.make_async_copy(out_v.at[pl.ds(skip, half - skip)],
                          out_hbm.at[pl.ds(base + skip, half - skip)],
                          sem_o0).wait()

  co1.wait()


@functools.partial(jax.jit, static_argnames=("chunk", "num_species"))
def _species_lookup(a, species, chunk, num_species):
  n = a.shape[0]
  mesh = plsc.VectorSubcoreMesh(core_axis_name="c", subcore_axis_name="s")
  body = functools.partial(_species_lookup_body, chunk, n, num_species)
  return pl.kernel(
      body,
      out_type=jax.ShapeDtypeStruct((n,), jnp.int32),
      mesh=mesh,
      scratch_types=[
          pltpu.VMEM((chunk,), jnp.int32),
          pltpu.VMEM((chunk,), jnp.int32),
          pltpu.VMEM((_LANES,), jnp.int32),
          pltpu.VMEM((_LANES,), jnp.int32),
          pltpu.SemaphoreType.DMA,
          pltpu.SemaphoreType.DMA,
          pltpu.SemaphoreType.DMA,
          pltpu.SemaphoreType.DMA,
          pltpu.SemaphoreType.DMA,
      ],
      compiler_params=pltpu.CompilerParams(needs_layout_passes=False, disable_bounds_checks=True, disable_semaphore_checks=True),
  )(a, species)


def kernel(atomic_numbers, species):
  n = atomic_numbers.shape[0]
  num_species = species.shape[0]

  chunk = -(-(-(-n // _NUM_WORKERS)) // (2 * _LANES)) * (2 * _LANES)
  assert n >= chunk and n % 8 == 0

  return _species_lookup(atomic_numbers.astype(jnp.int32),
                         species.astype(jnp.int32), chunk, num_species)


# speedup vs baseline: 1.0037x; 1.0037x over previous
import functools

import jax
import jax.numpy as jnp
from jax import lax
from jax.experimental import pallas as pl
from jax.experimental.pallas import tpu as pltpu
from jax.experimental.pallas import tpu_sc as plsc

_NUM_WORKERS = 32
_LANES = 16


def _species_lookup_body(chunk, n, num_species, a_hbm, species_hbm,
                         out_hbm, a_v, out_v, spec_v, lut_v,
                         sem_s, sem_a0, sem_a1, sem_o0, sem_o1):
  wid = lax.axis_index("s") * 2 + lax.axis_index("c")
  is_last = wid == _NUM_WORKERS - 1
  base = jnp.minimum(wid * chunk, n - chunk)

  half = chunk // 2
  cp0 = pltpu.async_copy(a_hbm.at[pl.ds(base, half)],
                         a_v.at[pl.ds(0, half)], sem_a0)
  cp1 = pltpu.async_copy(a_hbm.at[pl.ds(base + half, half)],
                         a_v.at[pl.ds(half, half)], sem_a1)

  spec_v[...] = jnp.zeros((_LANES,), jnp.int32)
  cps = pltpu.async_copy(species_hbm, spec_v.at[pl.ds(0, num_species)],
                         sem_s)
  cps.wait()
  lane = lax.iota(jnp.int32, _LANES)
  plsc.store_scatter(lut_v, [spec_v[...]], lane, mask=lane < num_species)

  cp0.wait()

  @plsc.parallel_loop(0, half, _LANES, unroll=8)
  def _(off):
    out_v[pl.ds(off, _LANES)] = plsc.load_gather(lut_v,
                                                 [a_v[pl.ds(off, _LANES)]])

  skip = chunk - (n - (_NUM_WORKERS - 1) * chunk)

  @pl.when(jnp.logical_not(is_last))
  def _():
    pltpu.async_copy(out_v.at[pl.ds(0, half)],
                     out_hbm.at[pl.ds(base, half)], sem_o0)

  @pl.when(is_last)
  def _():
    pltpu.async_copy(out_v.at[pl.ds(skip, half - skip)],
                     out_hbm.at[pl.ds(base + skip, half - skip)], sem_o0)

  cp1.wait()

  @plsc.parallel_loop(half, chunk, _LANES, unroll=8)
  def _(off):
    out_v[pl.ds(off, _LANES)] = plsc.load_gather(lut_v,
                                                 [a_v[pl.ds(off, _LANES)]])

  co1 = pltpu.async_copy(out_v.at[pl.ds(half, half)],
                         out_hbm.at[pl.ds(base + half, half)], sem_o1)

  @pl.when(jnp.logical_not(is_last))
  def _():
    pltpu.make_async_copy(out_v.at[pl.ds(0, half)],
                          out_hbm.at[pl.ds(base, half)], sem_o0).wait()

  @pl.when(is_last)
  def _():
    pltpu.make_async_copy(out_v.at[pl.ds(skip, half - skip)],
                          out_hbm.at[pl.ds(base + skip, half - skip)],
                          sem_o0).wait()

  co1.wait()


@functools.partial(jax.jit, static_argnames=("chunk", "num_species"))
def _species_lookup(a, species, chunk, num_species):
  n = a.shape[0]
  mesh = plsc.VectorSubcoreMesh(core_axis_name="c", subcore_axis_name="s")
  body = functools.partial(_species_lookup_body, chunk, n, num_species)
  return pl.kernel(
      body,
      out_type=jax.ShapeDtypeStruct((n,), jnp.int32),
      mesh=mesh,
      scratch_types=[
          pltpu.VMEM((chunk,), jnp.int32),
          pltpu.VMEM((chunk,), jnp.int32),
          pltpu.VMEM((_LANES,), jnp.int32),
          pltpu.VMEM((_LANES,), jnp.int32),
          pltpu.SemaphoreType.DMA,
          pltpu.SemaphoreType.DMA,
          pltpu.SemaphoreType.DMA,
          pltpu.SemaphoreType.DMA,
          pltpu.SemaphoreType.DMA,
      ],
      compiler_params=pltpu.CompilerParams(needs_layout_passes=False),
  )(a, species)


def kernel(atomic_numbers, species):
  n = atomic_numbers.shape[0]
  num_species = species.shape[0]

  chunk = -(-(-(-n // _NUM_WORKERS)) // (2 * _LANES)) * (2 * _LANES)
  assert n >= chunk and n % 8 == 0

  return _species_lookup(atomic_numbers.astype(jnp.int32),
                         species.astype(jnp.int32), chunk, num_species)
